# Initial kernel scaffold; baseline (speedup 1.0000x reference)
import jax
import jax.numpy as jnp
from jax.experimental import pallas as pl


def kernel(inputs):
    raise NotImplementedError("write your pallas kernel here")


# speedup vs baseline: 1.9092x; 1.9092x over previous
import jax
import jax.numpy as jnp
from jax.experimental import pallas as pl

_ITER_NUMS = 10
_NUM_CENTERS = 2


def _initialize(feature, key):
    B, K, D, T = feature.shape
    centers = jnp.zeros((B, K, D), dtype=feature.dtype)
    centers = centers.at[:, 0].set(feature[:, 0, :, 0])
    for idx in range(1, K):
        diff = feature[:, None] - centers[:, 0:idx + 1, None, :, None]
        dist = jnp.sum(jnp.abs(diff) ** 2, axis=-2)
        dist = jnp.min(dist, axis=1)
        dist = dist / jnp.sum(dist, axis=(1, 2), keepdims=True)
        dist_flat = dist.reshape(B, -1)
        prob = jnp.cumsum(dist_flat, axis=1)
        r = jax.random.uniform(jax.random.fold_in(key, idx), (B,), dtype=feature.dtype)
        cnt = jnp.sum((prob <= r[:, None]).astype(jnp.int32), axis=1)
        t_star = cnt - 1
        valid = t_star >= 0
        spk = jnp.clip(t_star // T, 0, K - 1)
        frame = jnp.clip(t_star % T, 0, T - 1)
        sel = feature[jnp.arange(B), spk, :, frame]
        new_c = jnp.where(valid[:, None], sel, centers[:, idx])
        centers = centers.at[:, idx].set(new_c)
    return centers


def _kmeans_body(x_ref, c0_ref, out_ref):
    x0 = x_ref[0, 0]
    x1 = x_ref[0, 1]
    n_total = jnp.float32(x0.shape[1] + x1.shape[1])

    def one_iter(_, centers):
        c0 = centers[:, 0:1]
        c1 = centers[:, 1:2]
        d00 = jnp.sum((x0 - c0) ** 2, axis=0, keepdims=True)
        d01 = jnp.sum((x0 - c1) ** 2, axis=0, keepdims=True)
        d10 = jnp.sum((x1 - c0) ** 2, axis=0, keepdims=True)
        d11 = jnp.sum((x1 - c1) ** 2, axis=0, keepdims=True)
        m0 = (d00 <= d01).astype(jnp.float32)
        m1 = (d10 <= d11).astype(jnp.float32)
        n0 = jnp.sum(m0) + jnp.sum(m1)
        s0 = (jnp.sum(x0 * m0, axis=1, keepdims=True)
              + jnp.sum(x1 * m1, axis=1, keepdims=True))
        s1 = (jnp.sum(x0 * (1.0 - m0), axis=1, keepdims=True)
              + jnp.sum(x1 * (1.0 - m1), axis=1, keepdims=True))
        new_c0 = s0 / (n0 + 1e-8)
        new_c1 = s1 / ((n_total - n0) + 1e-8)
        return jnp.concatenate([new_c0, new_c1], axis=1)

    centers = jax.lax.fori_loop(0, _ITER_NUMS, one_iter, c0_ref[0])
    out_ref[0] = centers


def kernel(inputs):
    B, K, D, T = inputs.shape
    centers0 = _initialize(inputs, jax.random.key(42))
    centers0_dk = centers0.transpose(0, 2, 1)
    out_dk = pl.pallas_call(
        _kmeans_body,
        grid=(B,),
        in_specs=[
            pl.BlockSpec((1, K, D, T), lambda b: (b, 0, 0, 0)),
            pl.BlockSpec((1, D, K), lambda b: (b, 0, 0)),
        ],
        out_specs=pl.BlockSpec((1, D, K), lambda b: (b, 0, 0)),
        out_shape=jax.ShapeDtypeStruct((B, D, K), jnp.float32),
    )(inputs, centers0_dk)
    return out_dk.transpose(0, 2, 1)


# speedup vs baseline: 2.1419x; 1.1219x over previous
import jax
import jax.numpy as jnp
from jax.experimental import pallas as pl

_ITER_NUMS = 10
_NUM_CENTERS = 2


def _initialize(feature, key):
    B, K, D, T = feature.shape
    centers = jnp.zeros((B, K, D), dtype=feature.dtype)
    centers = centers.at[:, 0].set(feature[:, 0, :, 0])
    for idx in range(1, K):
        diff = feature[:, None] - centers[:, 0:idx + 1, None, :, None]
        dist = jnp.sum(jnp.abs(diff) ** 2, axis=-2)
        dist = jnp.min(dist, axis=1)
        dist = dist / jnp.sum(dist, axis=(1, 2), keepdims=True)
        dist_flat = dist.reshape(B, -1)
        prob = jnp.cumsum(dist_flat, axis=1)
        r = jax.random.uniform(jax.random.fold_in(key, idx), (B,), dtype=feature.dtype)
        cnt = jnp.sum((prob <= r[:, None]).astype(jnp.int32), axis=1)
        t_star = cnt - 1
        valid = t_star >= 0
        spk = jnp.clip(t_star // T, 0, K - 1)
        frame = jnp.clip(t_star % T, 0, T - 1)
        sel = feature[jnp.arange(B), spk, :, frame]
        new_c = jnp.where(valid[:, None], sel, centers[:, idx])
        centers = centers.at[:, idx].set(new_c)
    return centers


def _kmeans_body(x_ref, c0_ref, out_ref):
    x0 = x_ref[0, 0]
    x1 = x_ref[0, 1]
    n_total = jnp.float32(x0.shape[1] + x1.shape[1])
    s_total = (jnp.sum(x0, axis=1, keepdims=True)
               + jnp.sum(x1, axis=1, keepdims=True))

    def one_iter(_, centers):
        c0 = centers[:, 0:1]
        c1 = centers[:, 1:2]
        d00 = jnp.sum((x0 - c0) ** 2, axis=0, keepdims=True)
        d01 = jnp.sum((x0 - c1) ** 2, axis=0, keepdims=True)
        d10 = jnp.sum((x1 - c0) ** 2, axis=0, keepdims=True)
        d11 = jnp.sum((x1 - c1) ** 2, axis=0, keepdims=True)
        m0 = (d00 <= d01).astype(jnp.float32)
        m1 = (d10 <= d11).astype(jnp.float32)
        n0 = jnp.sum(m0) + jnp.sum(m1)
        s0 = (jnp.sum(x0 * m0, axis=1, keepdims=True)
              + jnp.sum(x1 * m1, axis=1, keepdims=True))
        s1 = s_total - s0
        new_c0 = s0 / (n0 + 1e-8)
        new_c1 = s1 / ((n_total - n0) + 1e-8)
        return jnp.concatenate([new_c0, new_c1], axis=1)

    centers = jax.lax.fori_loop(0, _ITER_NUMS, one_iter, c0_ref[0])
    out_ref[0] = centers


def kernel(inputs):
    B, K, D, T = inputs.shape
    centers0 = _initialize(inputs, jax.random.key(42))
    centers0_dk = centers0.transpose(0, 2, 1)
    out_dk = pl.pallas_call(
        _kmeans_body,
        grid=(B,),
        in_specs=[
            pl.BlockSpec((1, K, D, T), lambda b: (b, 0, 0, 0)),
            pl.BlockSpec((1, D, K), lambda b: (b, 0, 0)),
        ],
        out_specs=pl.BlockSpec((1, D, K), lambda b: (b, 0, 0)),
        out_shape=jax.ShapeDtypeStruct((B, D, K), jnp.float32),
    )(inputs, centers0_dk)
    return out_dk.transpose(0, 2, 1)
